# baseline (speedup 1.0000x reference)
import functools

import jax
import jax.numpy as jnp
from jax import lax
from jax.experimental import pallas as pl
from jax.experimental.pallas import tpu as pltpu
from jax.experimental.pallas import tpu_sc as plsc

N = 10000
NH = N // 2
E = 320000
NP = 10112
NACC = 5016

_MESH = plsc.VectorSubcoreMesh(
    core_axis_name="c", subcore_axis_name="s", num_cores=2, num_subcores=16
)


def _f32z16():
    return jnp.zeros((16,), jnp.float32)


@functools.partial(
    pl.kernel,
    mesh=_MESH,
    out_type=[
        jax.ShapeDtypeStruct((NP, 128), jnp.float32),
        jax.ShapeDtypeStruct((2, NP), jnp.float32),
    ],
    scratch_types=[
        pltpu.VMEM((3, 128), jnp.int32),
        pltpu.VMEM((384, 128), jnp.float32),
        pltpu.VMEM((3, 128), jnp.int32),
        pltpu.VMEM((128,), jnp.float32),
        pltpu.VMEM((640,), jnp.float32),
        pltpu.VMEM_SHARED((NP,), jnp.float32),
        pltpu.SemaphoreType.DMA,
    ],
)
def _sc_emb_deg(x_hbm, a_hbm, tab_hbm, emb_hbm, deg_hbm,
                idxb, rows, dstb, ones, zb, degacc, gsem):
    c = lax.axis_index("c")
    s = lax.axis_index("s")
    w = s * 2 + c

    def fill_ones(i, _):
        ones[pl.ds(i * 16, 16)] = jnp.full((16,), 1.0, jnp.float32)
        return 0

    lax.fori_loop(0, 8, fill_ones, 0)

    def fill_z(i, _):
        zb[pl.ds(i * 16, 16)] = _f32z16()
        return 0

    lax.fori_loop(0, 40, fill_z, 0)

    @pl.when(s < 15)
    def _():
        pltpu.sync_copy(zb, degacc.at[pl.ds(s * 640, 640)])

    @pl.when(s == 15)
    def _():
        pltpu.sync_copy(zb.at[pl.ds(0, 512)], degacc.at[pl.ds(9600, 512)])

    plsc.subcore_barrier()

    ebase = w * (78 * 128)

    def deg_body(g, _):
        off = ebase + g * 384
        for j in range(3):
            pltpu.sync_copy(a_hbm.at[1].at[pl.ds(off + j * 128, 128)],
                            dstb.at[j])
        for j in range(3):
            pltpu.sync_copy(ones, degacc.at[dstb.at[j]], add=True)
        return 0

    lax.fori_loop(0, 26, deg_body, 0)

    @pl.when(w < 4)
    def _():
        off = (2496 + w) * 128
        pltpu.sync_copy(a_hbm.at[1].at[pl.ds(off, 128)], dstb.at[0])
        pltpu.sync_copy(ones, degacc.at[dstb.at[0]], add=True)

    plsc.subcore_barrier()

    @pl.when(s < 15)
    def _():
        pltpu.sync_copy(degacc.at[pl.ds(s * 640, 640)],
                        deg_hbm.at[c].at[pl.ds(s * 640, 640)])

    @pl.when(s == 15)
    def _():
        pltpu.sync_copy(degacc.at[pl.ds(9600, 512)],
                        deg_hbm.at[c].at[pl.ds(9600, 512)])

    @pl.when(w < 14)
    def _():
        rbase = w * 384
        for j in range(3):
            pltpu.sync_copy(x_hbm.at[pl.ds(rbase + j * 128, 128)],
                            idxb.at[j])
        cps = [
            pltpu.async_copy(tab_hbm.at[idxb.at[j]],
                             rows.at[pl.ds(j * 128, 128)], gsem)
            for j in range(3)
        ]
        for cp in cps:
            cp.wait()
        pltpu.sync_copy(rows, emb_hbm.at[pl.ds(rbase, 384)])

    @pl.when(w >= 14)
    def _():
        rbase = 42 * 128 + (w - 14) * 256
        for j in range(2):
            pltpu.sync_copy(x_hbm.at[pl.ds(rbase + j * 128, 128)],
                            idxb.at[j])
        cps = [
            pltpu.async_copy(tab_hbm.at[idxb.at[j]],
                             rows.at[pl.ds(j * 128, 128)], gsem)
            for j in range(2)
        ]
        for cp in cps:
            cp.wait()
        pltpu.sync_copy(rows.at[pl.ds(0, 256)],
                        emb_hbm.at[pl.ds(rbase, 256)])

    @pl.when(w == 31)
    def _():
        pltpu.sync_copy(x_hbm.at[pl.ds(9984, 128)], idxb.at[2])
        pltpu.async_copy(tab_hbm.at[idxb.at[2]],
                         rows.at[pl.ds(256, 128)], gsem).wait()
        pltpu.sync_copy(rows.at[pl.ds(256, 128)],
                        emb_hbm.at[pl.ds(9984, 128)])


_NPS = 156
_NGRP = 52


@functools.partial(
    pl.kernel,
    mesh=_MESH,
    out_type=[jax.ShapeDtypeStruct((2, NH, 128), jnp.float32)],
    scratch_types=[
        pltpu.VMEM((3, 128), jnp.int32),
        pltpu.VMEM((3, 128), jnp.int32),
        pltpu.VMEM((384, 128), jnp.float32),
        pltpu.VMEM((16, 128), jnp.float32),
        pltpu.VMEM_SHARED((NACC, 128), jnp.float32),
        pltpu.SemaphoreType.DMA,
    ],
)
def _sc_scatter(q_hbm, a_hbm, out_hbm, srcb, dstb, rows, zbuf, accum, gsem):
    c = lax.axis_index("c")
    s = lax.axis_index("s")
    cbase = c * NH
    trash = NH + s

    def fz(i, _):
        for j in range(8):
            zbuf[i, pl.ds(j * 16, 16)] = _f32z16()
        return 0

    lax.fori_loop(0, 16, fz, 0)

    def zero_rows(off, niter):
        def zc(i, _):
            pltpu.sync_copy(zbuf, accum.at[pl.ds(off + i * 16, 16)])
            return 0
        lax.fori_loop(0, niter, zc, 0)

    @pl.when(s < 15)
    def _():
        zero_rows(s * 320, 20)

    @pl.when(s == 15)
    def _():
        zero_rows(4800, 13)
        pltpu.sync_copy(zbuf.at[pl.ds(0, 8)], accum.at[pl.ds(5008, 8)])

    plsc.subcore_barrier()

    def remap(j):
        for k in range(8):
            d = dstb[j, pl.ds(k * 16, 16)]
            dl = d - cbase
            ok = (dl >= 0) & (dl < NH)
            dstb[j, pl.ds(k * 16, 16)] = jnp.where(ok, dl, trash)

    base = s * (_NPS * 128)

    def grp(g, _):
        off = base + g * 384
        for j in range(3):
            pltpu.sync_copy(a_hbm.at[0].at[pl.ds(off + j * 128, 128)],
                            srcb.at[j])
            pltpu.sync_copy(a_hbm.at[1].at[pl.ds(off + j * 128, 128)],
                            dstb.at[j])
        cps = [
            pltpu.async_copy(q_hbm.at[srcb.at[j]],
                             rows.at[pl.ds(j * 128, 128)], gsem)
            for j in range(3)
        ]
        for j in range(3):
            remap(j)
        for cp in cps:
            cp.wait()
        for j in range(3):
            pltpu.sync_copy(rows.at[pl.ds(j * 128, 128)],
                            accum.at[dstb.at[j]], add=True)
        return 0

    lax.fori_loop(0, _NGRP, grp, 0)

    @pl.when(s < 4)
    def _():
        off = (2496 + s) * 128
        pltpu.sync_copy(a_hbm.at[0].at[pl.ds(off, 128)], srcb.at[0])
        pltpu.sync_copy(a_hbm.at[1].at[pl.ds(off, 128)], dstb.at[0])
        cp = pltpu.async_copy(q_hbm.at[srcb.at[0]],
                              rows.at[pl.ds(0, 128)], gsem)
        remap(0)
        cp.wait()
        pltpu.sync_copy(rows.at[pl.ds(0, 128)],
                        accum.at[dstb.at[0]], add=True)

    plsc.subcore_barrier()

    @pl.when(s < 15)
    def _():
        pltpu.sync_copy(accum.at[pl.ds(s * 312, 312)],
                        out_hbm.at[c].at[pl.ds(s * 312, 312)])

    @pl.when(s == 15)
    def _():
        pltpu.sync_copy(accum.at[pl.ds(4680, 320)],
                        out_hbm.at[c].at[pl.ds(4680, 320)])


def _leaky(v):
    return jnp.where(v > 0, v, 0.01 * v)


def _dot(x, y):
    return lax.dot_general(x, y, (((1,), (0,)), ((), ())),
                           precision=lax.Precision.HIGHEST,
                           preferred_element_type=jnp.float32)


_BLK = 2000


def _prep_body(degb_ref, emb_ref, dinvb_ref, q1_ref):
    dv = lax.rsqrt(degb_ref[...])
    dinvb_ref[...] = dv
    q1_ref[...] = emb_ref[...] * dv


def _prep_call(degb, emb):
    return pl.pallas_call(
        _prep_body,
        grid=(N // _BLK,),
        in_specs=[
            pl.BlockSpec((_BLK, 128), lambda i: (i, 0)),
            pl.BlockSpec((_BLK, 128), lambda i: (i, 0)),
        ],
        out_specs=[
            pl.BlockSpec((_BLK, 128), lambda i: (i, 0)),
            pl.BlockSpec((_BLK, 128), lambda i: (i, 0)),
        ],
        out_shape=[
            jax.ShapeDtypeStruct((N, 128), jnp.float32),
            jax.ShapeDtypeStruct((N, 128), jnp.float32),
        ],
    )(degb, emb)


def _l1p1_body(s1_ref, q1_ref, dinvb_ref, w_ref, b_ref,
               t_ref, sum_ref, ssq_ref):
    agg = dinvb_ref[...] * (s1_ref[...] + q1_ref[...])
    t = _dot(agg, w_ref[...]) + b_ref[...]
    t_ref[...] = t

    @pl.when(pl.program_id(0) == 0)
    def _():
        sum_ref[...] = jnp.zeros_like(sum_ref)
        ssq_ref[...] = jnp.zeros_like(ssq_ref)

    sum_ref[...] += jnp.sum(t, 0, keepdims=True)
    ssq_ref[...] += jnp.sum(t * t, 0, keepdims=True)


def _l1p1_call(s1, q1, dinvb, W1, b1):
    return pl.pallas_call(
        _l1p1_body,
        grid=(N // _BLK,),
        in_specs=[
            pl.BlockSpec((_BLK, 128), lambda i: (i, 0)),
            pl.BlockSpec((_BLK, 128), lambda i: (i, 0)),
            pl.BlockSpec((_BLK, 128), lambda i: (i, 0)),
            pl.BlockSpec((128, 256), lambda i: (0, 0)),
            pl.BlockSpec((1, 256), lambda i: (0, 0)),
        ],
        out_specs=[
            pl.BlockSpec((_BLK, 256), lambda i: (i, 0)),
            pl.BlockSpec((1, 256), lambda i: (0, 0)),
            pl.BlockSpec((1, 256), lambda i: (0, 0)),
        ],
        out_shape=[
            jax.ShapeDtypeStruct((N, 256), jnp.float32),
            jax.ShapeDtypeStruct((1, 256), jnp.float32),
            jax.ShapeDtypeStruct((1, 256), jnp.float32),
        ],
    )(s1, q1, dinvb, W1, b1)


def _l1p2_body(t_ref, sum_ref, ssq_ref, g_ref, be_ref, dinvb_ref,
               q2a_ref, q2b_ref):
    m = sum_ref[...] * (1.0 / N)
    v = ssq_ref[...] * (1.0 / N) - m * m
    y = (t_ref[...] - m) * (lax.rsqrt(v + 1e-5) * g_ref[...]) + be_ref[...]
    q = _leaky(y)
    dv = dinvb_ref[...]
    q2a_ref[...] = q[:, :128] * dv
    q2b_ref[...] = q[:, 128:] * dv


def _l1p2_call(t1, s1, ssq1, g1, be1, dinvb):
    return pl.pallas_call(
        _l1p2_body,
        grid=(N // _BLK,),
        in_specs=[
            pl.BlockSpec((_BLK, 256), lambda i: (i, 0)),
            pl.BlockSpec((1, 256), lambda i: (0, 0)),
            pl.BlockSpec((1, 256), lambda i: (0, 0)),
            pl.BlockSpec((1, 256), lambda i: (0, 0)),
            pl.BlockSpec((1, 256), lambda i: (0, 0)),
            pl.BlockSpec((_BLK, 128), lambda i: (i, 0)),
        ],
        out_specs=[
            pl.BlockSpec((_BLK, 128), lambda i: (i, 0)),
            pl.BlockSpec((_BLK, 128), lambda i: (i, 0)),
        ],
        out_shape=[
            jax.ShapeDtypeStruct((N, 128), jnp.float32),
            jax.ShapeDtypeStruct((N, 128), jnp.float32),
        ],
    )(t1, s1, ssq1, g1, be1, dinvb)


def _l2p1_body(s2a_ref, s2b_ref, q2a_ref, q2b_ref, dinvb_ref, w_ref, b_ref,
               t_ref, sum_ref, ssq_ref):
    dv = dinvb_ref[...]
    t = (_dot(dv * (s2a_ref[...] + q2a_ref[...]), w_ref[0]) +
         _dot(dv * (s2b_ref[...] + q2b_ref[...]), w_ref[1]) + b_ref[...])
    t_ref[...] = t

    @pl.when(pl.program_id(0) == 0)
    def _():
        sum_ref[...] = jnp.zeros_like(sum_ref)
        ssq_ref[...] = jnp.zeros_like(ssq_ref)

    sum_ref[...] += jnp.sum(t, 0, keepdims=True)
    ssq_ref[...] += jnp.sum(t * t, 0, keepdims=True)


def _l2p1_call(s2a, s2b, q2a, q2b, dinvb, W2s, b2):
    return pl.pallas_call(
        _l2p1_body,
        grid=(N // _BLK,),
        in_specs=[
            pl.BlockSpec((_BLK, 128), lambda i: (i, 0)),
            pl.BlockSpec((_BLK, 128), lambda i: (i, 0)),
            pl.BlockSpec((_BLK, 128), lambda i: (i, 0)),
            pl.BlockSpec((_BLK, 128), lambda i: (i, 0)),
            pl.BlockSpec((_BLK, 128), lambda i: (i, 0)),
            pl.BlockSpec((2, 128, 512), lambda i: (0, 0, 0)),
            pl.BlockSpec((1, 512), lambda i: (0, 0)),
        ],
        out_specs=[
            pl.BlockSpec((_BLK, 512), lambda i: (i, 0)),
            pl.BlockSpec((1, 512), lambda i: (0, 0)),
            pl.BlockSpec((1, 512), lambda i: (0, 0)),
        ],
        out_shape=[
            jax.ShapeDtypeStruct((N, 512), jnp.float32),
            jax.ShapeDtypeStruct((1, 512), jnp.float32),
            jax.ShapeDtypeStruct((1, 512), jnp.float32),
        ],
    )(s2a, s2b, q2a, q2b, dinvb, W2s, b2)


def _l2p2_body(t_ref, sum_ref, ssq_ref, g_ref, be_ref, h_ref):
    m = sum_ref[...] * (1.0 / N)
    v = ssq_ref[...] * (1.0 / N) - m * m
    h_ref[...] = _leaky(
        (t_ref[...] - m) * (lax.rsqrt(v + 1e-5) * g_ref[...]) + be_ref[...])


def _l2p2_call(t2, s2, ssq2, g2, be2):
    return pl.pallas_call(
        _l2p2_body,
        grid=(N // _BLK,),
        in_specs=[
            pl.BlockSpec((_BLK, 512), lambda i: (i, 0)),
            pl.BlockSpec((1, 512), lambda i: (0, 0)),
            pl.BlockSpec((1, 512), lambda i: (0, 0)),
            pl.BlockSpec((1, 512), lambda i: (0, 0)),
            pl.BlockSpec((1, 512), lambda i: (0, 0)),
        ],
        out_specs=pl.BlockSpec((_BLK, 512), lambda i: (i, 0)),
        out_shape=jax.ShapeDtypeStruct((N, 512), jnp.float32),
    )(t2, s2, ssq2, g2, be2)


_MBLK = 1000


def _mlp_body(h_ref, w1_ref, b1_ref, w2_ref, b2_ref, o_ref):
    y = _leaky(_dot(h_ref[...], w1_ref[...]) + b1_ref[...])
    o_ref[...] = _dot(y, w2_ref[...]) + b2_ref[...]


def _mlp_call(h, D1W, D1b, D2W, D2b):
    return pl.pallas_call(
        _mlp_body,
        grid=(N // _MBLK,),
        in_specs=[
            pl.BlockSpec((_MBLK, 512), lambda i: (i, 0)),
            pl.BlockSpec((512, 4096), lambda i: (0, 0)),
            pl.BlockSpec((1, 4096), lambda i: (0, 0)),
            pl.BlockSpec((4096, 6), lambda i: (0, 0)),
            pl.BlockSpec((1, 6), lambda i: (0, 0)),
        ],
        out_specs=pl.BlockSpec((_MBLK, 6), lambda i: (i, 0)),
        out_shape=jax.ShapeDtypeStruct((N, 6), jnp.float32),
    )(h, D1W, D1b, D2W, D2b)


def _scatter(q, a):
    s = _sc_scatter(q, a)
    if isinstance(s, (list, tuple)):
        s = s[0]
    return s.reshape(N, 128)


@jax.jit
def _pipeline(x, a, emb_table, W1, b1, g1, be1, W2, b2, g2, be2,
              D1W, D1b, D2W, D2b):
    x = x.astype(jnp.int32)
    a = a.astype(jnp.int32)
    x_pad = jnp.concatenate([x, jnp.zeros((NP - N,), jnp.int32)])
    emb, deg_parts = _sc_emb_deg(x_pad, a, emb_table)
    deg = deg_parts[0, :N] + deg_parts[1, :N] + 1.0
    degb = jnp.broadcast_to(deg[:, None], (N, 128))
    dinvb, q1 = _prep_call(degb, emb[:N])
    s1 = _scatter(q1, a)
    t1, s1sum, s1ssq = _l1p1_call(s1, q1, dinvb, W1, b1.reshape(1, -1))
    q2a, q2b = _l1p2_call(t1, s1sum, s1ssq, g1.reshape(1, -1),
                          be1.reshape(1, -1), dinvb)
    s2a = _scatter(q2a, a)
    s2b = _scatter(q2b, a)
    t2, s2sum, s2ssq = _l2p1_call(s2a, s2b, q2a, q2b, dinvb,
                                  W2.reshape(2, 128, 512),
                                  b2.reshape(1, -1))
    h = _l2p2_call(t2, s2sum, s2ssq, g2.reshape(1, -1), be2.reshape(1, -1))
    return _mlp_call(h, D1W, D1b.reshape(1, -1), D2W, D2b.reshape(1, -1))


def kernel(x, a, emb_table, W1, b1, g1, be1, W2, b2, g2, be2,
           D1W, D1b, D2W, D2b):
    return _pipeline(x, a, emb_table, W1, b1, g1, be1, W2, b2, g2, be2,
                     D1W, D1b, D2W, D2b)


# speedup vs baseline: 1.3086x; 1.3086x over previous
import functools

import jax
import jax.numpy as jnp
from jax import lax
from jax.experimental import pallas as pl
from jax.experimental.pallas import tpu as pltpu
from jax.experimental.pallas import tpu_sc as plsc

N = 10000
NH = N // 2
E = 320000
NP = 10112
NACC = 5016

_MESH = plsc.VectorSubcoreMesh(
    core_axis_name="c", subcore_axis_name="s", num_cores=2, num_subcores=16
)


def _f32z16():
    return jnp.zeros((16,), jnp.float32)


@functools.partial(
    pl.kernel,
    mesh=_MESH,
    out_type=[
        jax.ShapeDtypeStruct((NP, 128), jnp.float32),
        jax.ShapeDtypeStruct((2, NP), jnp.float32),
    ],
    scratch_types=[
        pltpu.VMEM((3, 128), jnp.int32),
        pltpu.VMEM((384, 128), jnp.float32),
        pltpu.VMEM((3, 128), jnp.int32),
        pltpu.VMEM((128,), jnp.float32),
        pltpu.VMEM((640,), jnp.float32),
        pltpu.VMEM_SHARED((NP,), jnp.float32),
        pltpu.SemaphoreType.DMA,
    ],
)
def _sc_emb_deg(x_hbm, a_hbm, tab_hbm, emb_hbm, deg_hbm,
                idxb, rows, dstb, ones, zb, degacc, gsem):
    c = lax.axis_index("c")
    s = lax.axis_index("s")
    w = s * 2 + c

    def fill_ones(i, _):
        ones[pl.ds(i * 16, 16)] = jnp.full((16,), 1.0, jnp.float32)
        return 0

    lax.fori_loop(0, 8, fill_ones, 0)

    def fill_z(i, _):
        zb[pl.ds(i * 16, 16)] = _f32z16()
        return 0

    lax.fori_loop(0, 40, fill_z, 0)

    @pl.when(s < 15)
    def _():
        pltpu.sync_copy(zb, degacc.at[pl.ds(s * 640, 640)])

    @pl.when(s == 15)
    def _():
        pltpu.sync_copy(zb.at[pl.ds(0, 512)], degacc.at[pl.ds(9600, 512)])

    plsc.subcore_barrier()

    ebase = w * (78 * 128)

    def deg_body(g, _):
        off = ebase + g * 384
        for j in range(3):
            pltpu.sync_copy(a_hbm.at[1].at[pl.ds(off + j * 128, 128)],
                            dstb.at[j])
        for j in range(3):
            pltpu.sync_copy(ones, degacc.at[dstb.at[j]], add=True)
        return 0

    lax.fori_loop(0, 26, deg_body, 0)

    @pl.when(w < 4)
    def _():
        off = (2496 + w) * 128
        pltpu.sync_copy(a_hbm.at[1].at[pl.ds(off, 128)], dstb.at[0])
        pltpu.sync_copy(ones, degacc.at[dstb.at[0]], add=True)

    plsc.subcore_barrier()

    @pl.when(s < 15)
    def _():
        pltpu.sync_copy(degacc.at[pl.ds(s * 640, 640)],
                        deg_hbm.at[c].at[pl.ds(s * 640, 640)])

    @pl.when(s == 15)
    def _():
        pltpu.sync_copy(degacc.at[pl.ds(9600, 512)],
                        deg_hbm.at[c].at[pl.ds(9600, 512)])

    @pl.when(w < 14)
    def _():
        rbase = w * 384
        for j in range(3):
            pltpu.sync_copy(x_hbm.at[pl.ds(rbase + j * 128, 128)],
                            idxb.at[j])
        cps = [
            pltpu.async_copy(tab_hbm.at[idxb.at[j]],
                             rows.at[pl.ds(j * 128, 128)], gsem)
            for j in range(3)
        ]
        for cp in cps:
            cp.wait()
        pltpu.sync_copy(rows, emb_hbm.at[pl.ds(rbase, 384)])

    @pl.when(w >= 14)
    def _():
        rbase = 42 * 128 + (w - 14) * 256
        for j in range(2):
            pltpu.sync_copy(x_hbm.at[pl.ds(rbase + j * 128, 128)],
                            idxb.at[j])
        cps = [
            pltpu.async_copy(tab_hbm.at[idxb.at[j]],
                             rows.at[pl.ds(j * 128, 128)], gsem)
            for j in range(2)
        ]
        for cp in cps:
            cp.wait()
        pltpu.sync_copy(rows.at[pl.ds(0, 256)],
                        emb_hbm.at[pl.ds(rbase, 256)])

    @pl.when(w == 31)
    def _():
        pltpu.sync_copy(x_hbm.at[pl.ds(9984, 128)], idxb.at[2])
        pltpu.async_copy(tab_hbm.at[idxb.at[2]],
                         rows.at[pl.ds(256, 128)], gsem).wait()
        pltpu.sync_copy(rows.at[pl.ds(256, 128)],
                        emb_hbm.at[pl.ds(9984, 128)])


_NPS = 156
_NGRP = 52


@functools.partial(
    pl.kernel,
    mesh=_MESH,
    out_type=[jax.ShapeDtypeStruct((2, NH, 128), jnp.float32)],
    scratch_types=[
        pltpu.VMEM((3, 128), jnp.int32),
        pltpu.VMEM((3, 128), jnp.int32),
        pltpu.VMEM((384, 128), jnp.float32),
        pltpu.VMEM((16, 128), jnp.float32),
        pltpu.VMEM_SHARED((NACC, 128), jnp.float32),
        pltpu.SemaphoreType.DMA,
    ],
)
def _sc_scatter(q_hbm, a_hbm, out_hbm, srcb, dstb, rows, zbuf, accum, gsem):
    c = lax.axis_index("c")
    s = lax.axis_index("s")
    cbase = c * NH
    trash = NH + s

    def fz(i, _):
        for j in range(8):
            zbuf[i, pl.ds(j * 16, 16)] = _f32z16()
        return 0

    lax.fori_loop(0, 16, fz, 0)

    def zero_rows(off, niter):
        def zc(i, _):
            pltpu.sync_copy(zbuf, accum.at[pl.ds(off + i * 16, 16)])
            return 0
        lax.fori_loop(0, niter, zc, 0)

    @pl.when(s < 15)
    def _():
        zero_rows(s * 320, 20)

    @pl.when(s == 15)
    def _():
        zero_rows(4800, 13)
        pltpu.sync_copy(zbuf.at[pl.ds(0, 8)], accum.at[pl.ds(5008, 8)])

    plsc.subcore_barrier()

    def remap(j):
        for k in range(8):
            d = dstb[j, pl.ds(k * 16, 16)]
            dl = d - cbase
            ok = (dl >= 0) & (dl < NH)
            dstb[j, pl.ds(k * 16, 16)] = jnp.where(ok, dl, trash)

    base = s * (_NPS * 128)

    def grp(g, _):
        off = base + g * 384
        for j in range(3):
            pltpu.sync_copy(a_hbm.at[0].at[pl.ds(off + j * 128, 128)],
                            srcb.at[j])
            pltpu.sync_copy(a_hbm.at[1].at[pl.ds(off + j * 128, 128)],
                            dstb.at[j])
        cps = [
            pltpu.async_copy(q_hbm.at[srcb.at[j]],
                             rows.at[pl.ds(j * 128, 128)], gsem)
            for j in range(3)
        ]
        for j in range(3):
            remap(j)
        for cp in cps:
            cp.wait()
        for j in range(3):
            pltpu.sync_copy(rows.at[pl.ds(j * 128, 128)],
                            accum.at[dstb.at[j]], add=True)
        return 0

    lax.fori_loop(0, _NGRP, grp, 0)

    @pl.when(s < 4)
    def _():
        off = (2496 + s) * 128
        pltpu.sync_copy(a_hbm.at[0].at[pl.ds(off, 128)], srcb.at[0])
        pltpu.sync_copy(a_hbm.at[1].at[pl.ds(off, 128)], dstb.at[0])
        cp = pltpu.async_copy(q_hbm.at[srcb.at[0]],
                              rows.at[pl.ds(0, 128)], gsem)
        remap(0)
        cp.wait()
        pltpu.sync_copy(rows.at[pl.ds(0, 128)],
                        accum.at[dstb.at[0]], add=True)

    plsc.subcore_barrier()

    @pl.when(s < 15)
    def _():
        pltpu.sync_copy(accum.at[pl.ds(s * 312, 312)],
                        out_hbm.at[c].at[pl.ds(s * 312, 312)])

    @pl.when(s == 15)
    def _():
        pltpu.sync_copy(accum.at[pl.ds(4680, 320)],
                        out_hbm.at[c].at[pl.ds(4680, 320)])


def _leaky(v):
    return jnp.where(v > 0, v, 0.01 * v)


def _dot(x, y, precision=lax.Precision.HIGHEST):
    return lax.dot_general(x, y, (((1,), (0,)), ((), ())),
                           precision=precision,
                           preferred_element_type=jnp.float32)


_BLK = 2000


def _prep_body(degb_ref, emb_ref, dinvb_ref, q1_ref):
    dv = lax.rsqrt(degb_ref[...])
    dinvb_ref[...] = dv
    q1_ref[...] = emb_ref[...] * dv


def _prep_call(degb, emb):
    return pl.pallas_call(
        _prep_body,
        grid=(N // _BLK,),
        in_specs=[
            pl.BlockSpec((_BLK, 128), lambda i: (i, 0)),
            pl.BlockSpec((_BLK, 128), lambda i: (i, 0)),
        ],
        out_specs=[
            pl.BlockSpec((_BLK, 128), lambda i: (i, 0)),
            pl.BlockSpec((_BLK, 128), lambda i: (i, 0)),
        ],
        out_shape=[
            jax.ShapeDtypeStruct((N, 128), jnp.float32),
            jax.ShapeDtypeStruct((N, 128), jnp.float32),
        ],
    )(degb, emb)


def _l1p1_body(s1_ref, q1_ref, dinvb_ref, w_ref, b_ref,
               t_ref, sum_ref, ssq_ref):
    agg = dinvb_ref[...] * (s1_ref[...] + q1_ref[...])
    t = _dot(agg, w_ref[...]) + b_ref[...]
    t_ref[...] = t

    @pl.when(pl.program_id(0) == 0)
    def _():
        sum_ref[...] = jnp.zeros_like(sum_ref)
        ssq_ref[...] = jnp.zeros_like(ssq_ref)

    sum_ref[...] += jnp.sum(t, 0, keepdims=True)
    ssq_ref[...] += jnp.sum(t * t, 0, keepdims=True)


def _l1p1_call(s1, q1, dinvb, W1, b1):
    return pl.pallas_call(
        _l1p1_body,
        grid=(N // _BLK,),
        in_specs=[
            pl.BlockSpec((_BLK, 128), lambda i: (i, 0)),
            pl.BlockSpec((_BLK, 128), lambda i: (i, 0)),
            pl.BlockSpec((_BLK, 128), lambda i: (i, 0)),
            pl.BlockSpec((128, 256), lambda i: (0, 0)),
            pl.BlockSpec((1, 256), lambda i: (0, 0)),
        ],
        out_specs=[
            pl.BlockSpec((_BLK, 256), lambda i: (i, 0)),
            pl.BlockSpec((1, 256), lambda i: (0, 0)),
            pl.BlockSpec((1, 256), lambda i: (0, 0)),
        ],
        out_shape=[
            jax.ShapeDtypeStruct((N, 256), jnp.float32),
            jax.ShapeDtypeStruct((1, 256), jnp.float32),
            jax.ShapeDtypeStruct((1, 256), jnp.float32),
        ],
    )(s1, q1, dinvb, W1, b1)


def _l1p2_body(t_ref, sum_ref, ssq_ref, g_ref, be_ref, dinvb_ref,
               q2a_ref, q2b_ref):
    m = sum_ref[...] * (1.0 / N)
    v = ssq_ref[...] * (1.0 / N) - m * m
    y = (t_ref[...] - m) * (lax.rsqrt(v + 1e-5) * g_ref[...]) + be_ref[...]
    q = _leaky(y)
    dv = dinvb_ref[...]
    q2a_ref[...] = q[:, :128] * dv
    q2b_ref[...] = q[:, 128:] * dv


def _l1p2_call(t1, s1, ssq1, g1, be1, dinvb):
    return pl.pallas_call(
        _l1p2_body,
        grid=(N // _BLK,),
        in_specs=[
            pl.BlockSpec((_BLK, 256), lambda i: (i, 0)),
            pl.BlockSpec((1, 256), lambda i: (0, 0)),
            pl.BlockSpec((1, 256), lambda i: (0, 0)),
            pl.BlockSpec((1, 256), lambda i: (0, 0)),
            pl.BlockSpec((1, 256), lambda i: (0, 0)),
            pl.BlockSpec((_BLK, 128), lambda i: (i, 0)),
        ],
        out_specs=[
            pl.BlockSpec((_BLK, 128), lambda i: (i, 0)),
            pl.BlockSpec((_BLK, 128), lambda i: (i, 0)),
        ],
        out_shape=[
            jax.ShapeDtypeStruct((N, 128), jnp.float32),
            jax.ShapeDtypeStruct((N, 128), jnp.float32),
        ],
    )(t1, s1, ssq1, g1, be1, dinvb)


def _l2p1_body(s2a_ref, s2b_ref, q2a_ref, q2b_ref, dinvb_ref, w_ref, b_ref,
               t_ref, sum_ref, ssq_ref):
    dv = dinvb_ref[...]
    t = (_dot(dv * (s2a_ref[...] + q2a_ref[...]), w_ref[0]) +
         _dot(dv * (s2b_ref[...] + q2b_ref[...]), w_ref[1]) + b_ref[...])
    t_ref[...] = t

    @pl.when(pl.program_id(0) == 0)
    def _():
        sum_ref[...] = jnp.zeros_like(sum_ref)
        ssq_ref[...] = jnp.zeros_like(ssq_ref)

    sum_ref[...] += jnp.sum(t, 0, keepdims=True)
    ssq_ref[...] += jnp.sum(t * t, 0, keepdims=True)


def _l2p1_call(s2a, s2b, q2a, q2b, dinvb, W2s, b2):
    return pl.pallas_call(
        _l2p1_body,
        grid=(N // _BLK,),
        in_specs=[
            pl.BlockSpec((_BLK, 128), lambda i: (i, 0)),
            pl.BlockSpec((_BLK, 128), lambda i: (i, 0)),
            pl.BlockSpec((_BLK, 128), lambda i: (i, 0)),
            pl.BlockSpec((_BLK, 128), lambda i: (i, 0)),
            pl.BlockSpec((_BLK, 128), lambda i: (i, 0)),
            pl.BlockSpec((2, 128, 512), lambda i: (0, 0, 0)),
            pl.BlockSpec((1, 512), lambda i: (0, 0)),
        ],
        out_specs=[
            pl.BlockSpec((_BLK, 512), lambda i: (i, 0)),
            pl.BlockSpec((1, 512), lambda i: (0, 0)),
            pl.BlockSpec((1, 512), lambda i: (0, 0)),
        ],
        out_shape=[
            jax.ShapeDtypeStruct((N, 512), jnp.float32),
            jax.ShapeDtypeStruct((1, 512), jnp.float32),
            jax.ShapeDtypeStruct((1, 512), jnp.float32),
        ],
    )(s2a, s2b, q2a, q2b, dinvb, W2s, b2)


def _l2p2_body(t_ref, sum_ref, ssq_ref, g_ref, be_ref, h_ref):
    m = sum_ref[...] * (1.0 / N)
    v = ssq_ref[...] * (1.0 / N) - m * m
    h_ref[...] = _leaky(
        (t_ref[...] - m) * (lax.rsqrt(v + 1e-5) * g_ref[...]) + be_ref[...])


def _l2p2_call(t2, s2, ssq2, g2, be2):
    return pl.pallas_call(
        _l2p2_body,
        grid=(N // _BLK,),
        in_specs=[
            pl.BlockSpec((_BLK, 512), lambda i: (i, 0)),
            pl.BlockSpec((1, 512), lambda i: (0, 0)),
            pl.BlockSpec((1, 512), lambda i: (0, 0)),
            pl.BlockSpec((1, 512), lambda i: (0, 0)),
            pl.BlockSpec((1, 512), lambda i: (0, 0)),
        ],
        out_specs=pl.BlockSpec((_BLK, 512), lambda i: (i, 0)),
        out_shape=jax.ShapeDtypeStruct((N, 512), jnp.float32),
    )(t2, s2, ssq2, g2, be2)


_MBLK = 1000


def _mlp_body(h_ref, w1_ref, b1_ref, w2_ref, b2_ref, o_ref):
    y = _leaky(_dot(h_ref[...], w1_ref[...], None) + b1_ref[...])
    o_ref[...] = _dot(y, w2_ref[...], None) + b2_ref[...]


def _mlp_call(h, D1W, D1b, D2W, D2b):
    return pl.pallas_call(
        _mlp_body,
        grid=(N // _MBLK,),
        in_specs=[
            pl.BlockSpec((_MBLK, 512), lambda i: (i, 0)),
            pl.BlockSpec((512, 4096), lambda i: (0, 0)),
            pl.BlockSpec((1, 4096), lambda i: (0, 0)),
            pl.BlockSpec((4096, 6), lambda i: (0, 0)),
            pl.BlockSpec((1, 6), lambda i: (0, 0)),
        ],
        out_specs=pl.BlockSpec((_MBLK, 6), lambda i: (i, 0)),
        out_shape=jax.ShapeDtypeStruct((N, 6), jnp.float32),
    )(h, D1W, D1b, D2W, D2b)


def _scatter(q, a):
    s = _sc_scatter(q, a)
    if isinstance(s, (list, tuple)):
        s = s[0]
    return s.reshape(N, 128)


@jax.jit
def _pipeline(x, a, emb_table, W1, b1, g1, be1, W2, b2, g2, be2,
              D1W, D1b, D2W, D2b):
    x = x.astype(jnp.int32)
    a = a.astype(jnp.int32)
    x_pad = jnp.concatenate([x, jnp.zeros((NP - N,), jnp.int32)])
    emb, deg_parts = _sc_emb_deg(x_pad, a, emb_table)
    deg = deg_parts[0, :N] + deg_parts[1, :N] + 1.0
    degb = jnp.broadcast_to(deg[:, None], (N, 128))
    dinvb, q1 = _prep_call(degb, emb[:N])
    s1 = _scatter(q1, a)
    t1, s1sum, s1ssq = _l1p1_call(s1, q1, dinvb, W1, b1.reshape(1, -1))
    q2a, q2b = _l1p2_call(t1, s1sum, s1ssq, g1.reshape(1, -1),
                          be1.reshape(1, -1), dinvb)
    s2a = _scatter(q2a, a)
    s2b = _scatter(q2b, a)
    t2, s2sum, s2ssq = _l2p1_call(s2a, s2b, q2a, q2b, dinvb,
                                  W2.reshape(2, 128, 512),
                                  b2.reshape(1, -1))
    h = _l2p2_call(t2, s2sum, s2ssq, g2.reshape(1, -1), be2.reshape(1, -1))
    return _mlp_call(h, D1W, D1b.reshape(1, -1), D2W, D2b.reshape(1, -1))


def kernel(x, a, emb_table, W1, b1, g1, be1, W2, b2, g2, be2,
           D1W, D1b, D2W, D2b):
    return _pipeline(x, a, emb_table, W1, b1, g1, be1, W2, b2, g2, be2,
                     D1W, D1b, D2W, D2b)


# speedup vs baseline: 2.0835x; 1.5922x over previous
import functools

import jax
import jax.numpy as jnp
from jax import lax
from jax.experimental import pallas as pl
from jax.experimental.pallas import tpu as pltpu
from jax.experimental.pallas import tpu_sc as plsc

N = 10000
NH = N // 2
E = 320000
NP = 10112
NACC = 5016

_MESH = plsc.VectorSubcoreMesh(
    core_axis_name="c", subcore_axis_name="s", num_cores=2, num_subcores=16
)


def _f32z16():
    return jnp.zeros((16,), jnp.float32)


@functools.partial(
    pl.kernel,
    mesh=_MESH,
    out_type=[
        jax.ShapeDtypeStruct((NP, 128), jnp.float32),
        jax.ShapeDtypeStruct((2, NP), jnp.float32),
    ],
    scratch_types=[
        pltpu.VMEM((3, 128), jnp.int32),
        pltpu.VMEM((384, 128), jnp.float32),
        pltpu.VMEM((3, 128), jnp.int32),
        pltpu.VMEM((128,), jnp.float32),
        pltpu.VMEM((640,), jnp.float32),
        pltpu.VMEM_SHARED((NP,), jnp.float32),
        pltpu.SemaphoreType.DMA,
    ],
)
def _sc_emb_deg(x_hbm, a_hbm, tab_hbm, emb_hbm, deg_hbm,
                idxb, rows, dstb, ones, zb, degacc, gsem):
    c = lax.axis_index("c")
    s = lax.axis_index("s")
    w = s * 2 + c

    def fill_ones(i, _):
        ones[pl.ds(i * 16, 16)] = jnp.full((16,), 1.0, jnp.float32)
        return 0

    lax.fori_loop(0, 8, fill_ones, 0)

    def fill_z(i, _):
        zb[pl.ds(i * 16, 16)] = _f32z16()
        return 0

    lax.fori_loop(0, 40, fill_z, 0)

    @pl.when(s < 15)
    def _():
        pltpu.sync_copy(zb, degacc.at[pl.ds(s * 640, 640)])

    @pl.when(s == 15)
    def _():
        pltpu.sync_copy(zb.at[pl.ds(0, 512)], degacc.at[pl.ds(9600, 512)])

    plsc.subcore_barrier()

    ebase = w * (78 * 128)

    def deg_body(g, _):
        off = ebase + g * 384
        for j in range(3):
            pltpu.sync_copy(a_hbm.at[1].at[pl.ds(off + j * 128, 128)],
                            dstb.at[j])
        for j in range(3):
            pltpu.sync_copy(ones, degacc.at[dstb.at[j]], add=True)
        return 0

    lax.fori_loop(0, 26, deg_body, 0)

    @pl.when(w < 4)
    def _():
        off = (2496 + w) * 128
        pltpu.sync_copy(a_hbm.at[1].at[pl.ds(off, 128)], dstb.at[0])
        pltpu.sync_copy(ones, degacc.at[dstb.at[0]], add=True)

    plsc.subcore_barrier()

    @pl.when(s < 15)
    def _():
        pltpu.sync_copy(degacc.at[pl.ds(s * 640, 640)],
                        deg_hbm.at[c].at[pl.ds(s * 640, 640)])

    @pl.when(s == 15)
    def _():
        pltpu.sync_copy(degacc.at[pl.ds(9600, 512)],
                        deg_hbm.at[c].at[pl.ds(9600, 512)])

    @pl.when(w < 14)
    def _():
        rbase = w * 384
        for j in range(3):
            pltpu.sync_copy(x_hbm.at[pl.ds(rbase + j * 128, 128)],
                            idxb.at[j])
        cps = [
            pltpu.async_copy(tab_hbm.at[idxb.at[j]],
                             rows.at[pl.ds(j * 128, 128)], gsem)
            for j in range(3)
        ]
        for cp in cps:
            cp.wait()
        pltpu.sync_copy(rows, emb_hbm.at[pl.ds(rbase, 384)])

    @pl.when(w >= 14)
    def _():
        rbase = 42 * 128 + (w - 14) * 256
        for j in range(2):
            pltpu.sync_copy(x_hbm.at[pl.ds(rbase + j * 128, 128)],
                            idxb.at[j])
        cps = [
            pltpu.async_copy(tab_hbm.at[idxb.at[j]],
                             rows.at[pl.ds(j * 128, 128)], gsem)
            for j in range(2)
        ]
        for cp in cps:
            cp.wait()
        pltpu.sync_copy(rows.at[pl.ds(0, 256)],
                        emb_hbm.at[pl.ds(rbase, 256)])

    @pl.when(w == 31)
    def _():
        pltpu.sync_copy(x_hbm.at[pl.ds(9984, 128)], idxb.at[2])
        pltpu.async_copy(tab_hbm.at[idxb.at[2]],
                         rows.at[pl.ds(256, 128)], gsem).wait()
        pltpu.sync_copy(rows.at[pl.ds(256, 128)],
                        emb_hbm.at[pl.ds(9984, 128)])


_NPS = 156
_NGRP = 78


@functools.partial(
    pl.kernel,
    mesh=_MESH,
    out_type=[jax.ShapeDtypeStruct((2, NH, 128), jnp.float32)],
    scratch_types=[
        pltpu.VMEM((256,), jnp.int32),
        pltpu.VMEM((256,), jnp.int32),
        pltpu.VMEM((256,), jnp.int32),
        pltpu.VMEM((256,), jnp.int32),
        pltpu.VMEM((2, 128), jnp.int32),
        pltpu.VMEM((2, 128), jnp.int32),
        pltpu.VMEM((256, 128), jnp.float32),
        pltpu.VMEM((256, 128), jnp.float32),
        pltpu.VMEM((32, 128), jnp.float32),
        pltpu.VMEM_SHARED((NACC, 128), jnp.float32),
        pltpu.SemaphoreType.DMA,
        pltpu.SemaphoreType.DMA,
    ],
)
def _sc_scatter(q_hbm, a_hbm, out_hbm, srcA, srcB, dstA1, dstB1,
                dstbA, dstbB, rowsA, rowsB, zbuf, accum, gsemA, gsemB):
    c = lax.axis_index("c")
    s = lax.axis_index("s")
    cbase = c * NH
    trash = NH + s

    def fz(i, _):
        for j in range(8):
            zbuf[i, pl.ds(j * 16, 16)] = _f32z16()
        return 0

    lax.fori_loop(0, 32, fz, 0)

    @pl.when(s < 15)
    def _():
        def zc(i, _):
            pltpu.sync_copy(zbuf, accum.at[pl.ds(s * 320 + i * 32, 32)])
            return 0
        lax.fori_loop(0, 10, zc, 0)

    @pl.when(s == 15)
    def _():
        def zc(i, _):
            pltpu.sync_copy(zbuf, accum.at[pl.ds(4800 + i * 32, 32)])
            return 0
        lax.fori_loop(0, 6, zc, 0)
        pltpu.sync_copy(zbuf.at[pl.ds(0, 24)], accum.at[pl.ds(4992, 24)])

    plsc.subcore_barrier()

    def remap(dst1, dstb):
        for j in range(2):
            for k in range(8):
                d = dst1[pl.ds(j * 128 + k * 16, 16)]
                dl = d - cbase
                ok = (dl >= 0) & (dl < NH)
                dstb[j, pl.ds(k * 16, 16)] = jnp.where(ok, dl, trash)

    base = s * (_NPS * 128)

    def load_and_fire(g, srcb, dst1, rowsb, sem):
        off = base + g * 256
        pltpu.sync_copy(a_hbm.at[0].at[pl.ds(off, 256)], srcb)
        pltpu.sync_copy(a_hbm.at[1].at[pl.ds(off, 256)], dst1)
        for j in range(2):
            pltpu.async_copy(q_hbm.at[srcb.at[pl.ds(j * 128, 128)]],
                             rowsb.at[pl.ds(j * 128, 128)], sem)

    def drain(srcb, rowsb, sem):
        for j in range(2):
            pltpu.make_async_copy(q_hbm.at[srcb.at[pl.ds(j * 128, 128)]],
                                  rowsb.at[pl.ds(j * 128, 128)], sem).wait()

    def scatter(rowsb, dstb):
        for j in range(2):
            pltpu.sync_copy(rowsb.at[pl.ds(j * 128, 128)],
                            accum.at[dstb.at[j]], add=True)

    load_and_fire(0, srcA, dstA1, rowsA, gsemA)

    def body(gg, _):
        g1 = 2 * gg + 1
        load_and_fire(g1, srcB, dstB1, rowsB, gsemB)
        drain(srcA, rowsA, gsemA)
        remap(dstA1, dstbA)
        scatter(rowsA, dstbA)

        @pl.when(gg < _NGRP // 2 - 1)
        def _():
            load_and_fire(g1 + 1, srcA, dstA1, rowsA, gsemA)

        drain(srcB, rowsB, gsemB)
        remap(dstB1, dstbB)
        scatter(rowsB, dstbB)
        return 0

    lax.fori_loop(0, _NGRP // 2, body, 0)

    @pl.when(s < 4)
    def _():
        off = (2496 + s) * 128
        pltpu.sync_copy(a_hbm.at[0].at[pl.ds(off, 128)],
                        srcA.at[pl.ds(0, 128)])
        pltpu.sync_copy(a_hbm.at[1].at[pl.ds(off, 128)],
                        dstA1.at[pl.ds(0, 128)])
        cp = pltpu.async_copy(q_hbm.at[srcA.at[pl.ds(0, 128)]],
                              rowsA.at[pl.ds(0, 128)], gsemA)
        for k in range(8):
            d = dstA1[pl.ds(k * 16, 16)]
            dl = d - cbase
            ok = (dl >= 0) & (dl < NH)
            dstbA[0, pl.ds(k * 16, 16)] = jnp.where(ok, dl, trash)
        cp.wait()
        pltpu.sync_copy(rowsA.at[pl.ds(0, 128)],
                        accum.at[dstbA.at[0]], add=True)

    plsc.subcore_barrier()

    @pl.when(s < 15)
    def _():
        pltpu.sync_copy(accum.at[pl.ds(s * 312, 312)],
                        out_hbm.at[c].at[pl.ds(s * 312, 312)])

    @pl.when(s == 15)
    def _():
        pltpu.sync_copy(accum.at[pl.ds(4680, 320)],
                        out_hbm.at[c].at[pl.ds(4680, 320)])


def _leaky(v):
    return jnp.where(v > 0, v, 0.01 * v)


def _dot(x, y, precision=lax.Precision.HIGHEST):
    return lax.dot_general(x, y, (((1,), (0,)), ((), ())),
                           precision=precision,
                           preferred_element_type=jnp.float32)


_BLK = 2000


def _prep_body(degb_ref, emb_ref, dinvb_ref, q1_ref):
    dv = lax.rsqrt(degb_ref[...])
    dinvb_ref[...] = dv
    q1_ref[...] = emb_ref[...] * dv


def _prep_call(degb, emb):
    return pl.pallas_call(
        _prep_body,
        grid=(N // _BLK,),
        in_specs=[
            pl.BlockSpec((_BLK, 128), lambda i: (i, 0)),
            pl.BlockSpec((_BLK, 128), lambda i: (i, 0)),
        ],
        out_specs=[
            pl.BlockSpec((_BLK, 128), lambda i: (i, 0)),
            pl.BlockSpec((_BLK, 128), lambda i: (i, 0)),
        ],
        out_shape=[
            jax.ShapeDtypeStruct((N, 128), jnp.float32),
            jax.ShapeDtypeStruct((N, 128), jnp.float32),
        ],
    )(degb, emb)


def _l1p1_body(s1_ref, q1_ref, dinvb_ref, w_ref, b_ref,
               t_ref, sum_ref, ssq_ref):
    agg = dinvb_ref[...] * (s1_ref[...] + q1_ref[...])
    t = _dot(agg, w_ref[...]) + b_ref[...]
    t_ref[...] = t

    @pl.when(pl.program_id(0) == 0)
    def _():
        sum_ref[...] = jnp.zeros_like(sum_ref)
        ssq_ref[...] = jnp.zeros_like(ssq_ref)

    sum_ref[...] += jnp.sum(t, 0, keepdims=True)
    ssq_ref[...] += jnp.sum(t * t, 0, keepdims=True)


def _l1p1_call(s1, q1, dinvb, W1, b1):
    return pl.pallas_call(
        _l1p1_body,
        grid=(N // _BLK,),
        in_specs=[
            pl.BlockSpec((_BLK, 128), lambda i: (i, 0)),
            pl.BlockSpec((_BLK, 128), lambda i: (i, 0)),
            pl.BlockSpec((_BLK, 128), lambda i: (i, 0)),
            pl.BlockSpec((128, 256), lambda i: (0, 0)),
            pl.BlockSpec((1, 256), lambda i: (0, 0)),
        ],
        out_specs=[
            pl.BlockSpec((_BLK, 256), lambda i: (i, 0)),
            pl.BlockSpec((1, 256), lambda i: (0, 0)),
            pl.BlockSpec((1, 256), lambda i: (0, 0)),
        ],
        out_shape=[
            jax.ShapeDtypeStruct((N, 256), jnp.float32),
            jax.ShapeDtypeStruct((1, 256), jnp.float32),
            jax.ShapeDtypeStruct((1, 256), jnp.float32),
        ],
    )(s1, q1, dinvb, W1, b1)


def _l1p2_body(t_ref, sum_ref, ssq_ref, g_ref, be_ref, dinvb_ref,
               q2a_ref, q2b_ref):
    m = sum_ref[...] * (1.0 / N)
    v = ssq_ref[...] * (1.0 / N) - m * m
    y = (t_ref[...] - m) * (lax.rsqrt(v + 1e-5) * g_ref[...]) + be_ref[...]
    q = _leaky(y)
    dv = dinvb_ref[...]
    q2a_ref[...] = q[:, :128] * dv
    q2b_ref[...] = q[:, 128:] * dv


def _l1p2_call(t1, s1, ssq1, g1, be1, dinvb):
    return pl.pallas_call(
        _l1p2_body,
        grid=(N // _BLK,),
        in_specs=[
            pl.BlockSpec((_BLK, 256), lambda i: (i, 0)),
            pl.BlockSpec((1, 256), lambda i: (0, 0)),
            pl.BlockSpec((1, 256), lambda i: (0, 0)),
            pl.BlockSpec((1, 256), lambda i: (0, 0)),
            pl.BlockSpec((1, 256), lambda i: (0, 0)),
            pl.BlockSpec((_BLK, 128), lambda i: (i, 0)),
        ],
        out_specs=[
            pl.BlockSpec((_BLK, 128), lambda i: (i, 0)),
            pl.BlockSpec((_BLK, 128), lambda i: (i, 0)),
        ],
        out_shape=[
            jax.ShapeDtypeStruct((N, 128), jnp.float32),
            jax.ShapeDtypeStruct((N, 128), jnp.float32),
        ],
    )(t1, s1, ssq1, g1, be1, dinvb)


def _l2p1_body(s2a_ref, s2b_ref, q2a_ref, q2b_ref, dinvb_ref, w_ref, b_ref,
               t_ref, sum_ref, ssq_ref):
    dv = dinvb_ref[...]
    t = (_dot(dv * (s2a_ref[...] + q2a_ref[...]), w_ref[0]) +
         _dot(dv * (s2b_ref[...] + q2b_ref[...]), w_ref[1]) + b_ref[...])
    t_ref[...] = t

    @pl.when(pl.program_id(0) == 0)
    def _():
        sum_ref[...] = jnp.zeros_like(sum_ref)
        ssq_ref[...] = jnp.zeros_like(ssq_ref)

    sum_ref[...] += jnp.sum(t, 0, keepdims=True)
    ssq_ref[...] += jnp.sum(t * t, 0, keepdims=True)


def _l2p1_call(s2a, s2b, q2a, q2b, dinvb, W2s, b2):
    return pl.pallas_call(
        _l2p1_body,
        grid=(N // _BLK,),
        in_specs=[
            pl.BlockSpec((_BLK, 128), lambda i: (i, 0)),
            pl.BlockSpec((_BLK, 128), lambda i: (i, 0)),
            pl.BlockSpec((_BLK, 128), lambda i: (i, 0)),
            pl.BlockSpec((_BLK, 128), lambda i: (i, 0)),
            pl.BlockSpec((_BLK, 128), lambda i: (i, 0)),
            pl.BlockSpec((2, 128, 512), lambda i: (0, 0, 0)),
            pl.BlockSpec((1, 512), lambda i: (0, 0)),
        ],
        out_specs=[
            pl.BlockSpec((_BLK, 512), lambda i: (i, 0)),
            pl.BlockSpec((1, 512), lambda i: (0, 0)),
            pl.BlockSpec((1, 512), lambda i: (0, 0)),
        ],
        out_shape=[
            jax.ShapeDtypeStruct((N, 512), jnp.float32),
            jax.ShapeDtypeStruct((1, 512), jnp.float32),
            jax.ShapeDtypeStruct((1, 512), jnp.float32),
        ],
    )(s2a, s2b, q2a, q2b, dinvb, W2s, b2)


def _l2p2_body(t_ref, sum_ref, ssq_ref, g_ref, be_ref, h_ref):
    m = sum_ref[...] * (1.0 / N)
    v = ssq_ref[...] * (1.0 / N) - m * m
    h_ref[...] = _leaky(
        (t_ref[...] - m) * (lax.rsqrt(v + 1e-5) * g_ref[...]) + be_ref[...])


def _l2p2_call(t2, s2, ssq2, g2, be2):
    return pl.pallas_call(
        _l2p2_body,
        grid=(N // _BLK,),
        in_specs=[
            pl.BlockSpec((_BLK, 512), lambda i: (i, 0)),
            pl.BlockSpec((1, 512), lambda i: (0, 0)),
            pl.BlockSpec((1, 512), lambda i: (0, 0)),
            pl.BlockSpec((1, 512), lambda i: (0, 0)),
            pl.BlockSpec((1, 512), lambda i: (0, 0)),
        ],
        out_specs=pl.BlockSpec((_BLK, 512), lambda i: (i, 0)),
        out_shape=jax.ShapeDtypeStruct((N, 512), jnp.float32),
    )(t2, s2, ssq2, g2, be2)


_MBLK = 1000


def _mlp_body(h_ref, w1_ref, b1_ref, w2_ref, b2_ref, o_ref):
    y = _leaky(_dot(h_ref[...], w1_ref[...], None) + b1_ref[...])
    o_ref[...] = _dot(y, w2_ref[...], None) + b2_ref[...]


def _mlp_call(h, D1W, D1b, D2W, D2b):
    return pl.pallas_call(
        _mlp_body,
        grid=(N // _MBLK,),
        in_specs=[
            pl.BlockSpec((_MBLK, 512), lambda i: (i, 0)),
            pl.BlockSpec((512, 4096), lambda i: (0, 0)),
            pl.BlockSpec((1, 4096), lambda i: (0, 0)),
            pl.BlockSpec((4096, 6), lambda i: (0, 0)),
            pl.BlockSpec((1, 6), lambda i: (0, 0)),
        ],
        out_specs=pl.BlockSpec((_MBLK, 6), lambda i: (i, 0)),
        out_shape=jax.ShapeDtypeStruct((N, 6), jnp.float32),
    )(h, D1W, D1b, D2W, D2b)


def _scatter(q, a):
    s = _sc_scatter(q, a)
    if isinstance(s, (list, tuple)):
        s = s[0]
    return s.reshape(N, 128)


@jax.jit
def _pipeline(x, a, emb_table, W1, b1, g1, be1, W2, b2, g2, be2,
              D1W, D1b, D2W, D2b):
    x = x.astype(jnp.int32)
    a = a.astype(jnp.int32)
    x_pad = jnp.concatenate([x, jnp.zeros((NP - N,), jnp.int32)])
    emb, deg_parts = _sc_emb_deg(x_pad, a, emb_table)
    deg = deg_parts[0, :N] + deg_parts[1, :N] + 1.0
    degb = jnp.broadcast_to(deg[:, None], (N, 128))
    dinvb, q1 = _prep_call(degb, emb[:N])
    s1 = _scatter(q1, a)
    t1, s1sum, s1ssq = _l1p1_call(s1, q1, dinvb, W1, b1.reshape(1, -1))
    q2a, q2b = _l1p2_call(t1, s1sum, s1ssq, g1.reshape(1, -1),
                          be1.reshape(1, -1), dinvb)
    s2a = _scatter(q2a, a)
    s2b = _scatter(q2b, a)
    t2, s2sum, s2ssq = _l2p1_call(s2a, s2b, q2a, q2b, dinvb,
                                  W2.reshape(2, 128, 512),
                                  b2.reshape(1, -1))
    h = _l2p2_call(t2, s2sum, s2ssq, g2.reshape(1, -1), be2.reshape(1, -1))
    return _mlp_call(h, D1W, D1b.reshape(1, -1), D2W, D2b.reshape(1, -1))


def kernel(x, a, emb_table, W1, b1, g1, be1, W2, b2, g2, be2,
           D1W, D1b, D2W, D2b):
    return _pipeline(x, a, emb_table, W1, b1, g1, be1, W2, b2, g2, be2,
                     D1W, D1b, D2W, D2b)


# speedup vs baseline: 2.1019x; 1.0088x over previous
import functools

import jax
import jax.numpy as jnp
from jax import lax
from jax.experimental import pallas as pl
from jax.experimental.pallas import tpu as pltpu
from jax.experimental.pallas import tpu_sc as plsc

N = 10000
NH = N // 2
E = 320000
NP = 10112
NACC = 5016

_MESH = plsc.VectorSubcoreMesh(
    core_axis_name="c", subcore_axis_name="s", num_cores=2, num_subcores=16
)


def _f32z16():
    return jnp.zeros((16,), jnp.float32)


@functools.partial(
    pl.kernel,
    mesh=_MESH,
    out_type=[
        jax.ShapeDtypeStruct((NP, 128), jnp.float32),
        jax.ShapeDtypeStruct((2, NP), jnp.float32),
    ],
    scratch_types=[
        pltpu.VMEM((3, 128), jnp.int32),
        pltpu.VMEM((384, 128), jnp.float32),
        pltpu.VMEM((3, 128), jnp.int32),
        pltpu.VMEM((128,), jnp.float32),
        pltpu.VMEM((640,), jnp.float32),
        pltpu.VMEM_SHARED((NP,), jnp.float32),
        pltpu.SemaphoreType.DMA,
    ],
)
def _sc_emb_deg(x_hbm, a_hbm, tab_hbm, emb_hbm, deg_hbm,
                idxb, rows, dstb, ones, zb, degacc, gsem):
    c = lax.axis_index("c")
    s = lax.axis_index("s")
    w = s * 2 + c

    def fill_ones(i, _):
        ones[pl.ds(i * 16, 16)] = jnp.full((16,), 1.0, jnp.float32)
        return 0

    lax.fori_loop(0, 8, fill_ones, 0)

    def fill_z(i, _):
        zb[pl.ds(i * 16, 16)] = _f32z16()
        return 0

    lax.fori_loop(0, 40, fill_z, 0)

    @pl.when(s < 15)
    def _():
        pltpu.sync_copy(zb, degacc.at[pl.ds(s * 640, 640)])

    @pl.when(s == 15)
    def _():
        pltpu.sync_copy(zb.at[pl.ds(0, 512)], degacc.at[pl.ds(9600, 512)])

    plsc.subcore_barrier()

    ebase = w * (78 * 128)

    def deg_body(g, _):
        off = ebase + g * 384
        for j in range(3):
            pltpu.sync_copy(a_hbm.at[1].at[pl.ds(off + j * 128, 128)],
                            dstb.at[j])
        for j in range(3):
            pltpu.sync_copy(ones, degacc.at[dstb.at[j]], add=True)
        return 0

    lax.fori_loop(0, 26, deg_body, 0)

    @pl.when(w < 4)
    def _():
        off = (2496 + w) * 128
        pltpu.sync_copy(a_hbm.at[1].at[pl.ds(off, 128)], dstb.at[0])
        pltpu.sync_copy(ones, degacc.at[dstb.at[0]], add=True)

    plsc.subcore_barrier()

    @pl.when(s < 15)
    def _():
        pltpu.sync_copy(degacc.at[pl.ds(s * 640, 640)],
                        deg_hbm.at[c].at[pl.ds(s * 640, 640)])

    @pl.when(s == 15)
    def _():
        pltpu.sync_copy(degacc.at[pl.ds(9600, 512)],
                        deg_hbm.at[c].at[pl.ds(9600, 512)])

    @pl.when(w < 14)
    def _():
        rbase = w * 384
        for j in range(3):
            pltpu.sync_copy(x_hbm.at[pl.ds(rbase + j * 128, 128)],
                            idxb.at[j])
        cps = [
            pltpu.async_copy(tab_hbm.at[idxb.at[j]],
                             rows.at[pl.ds(j * 128, 128)], gsem)
            for j in range(3)
        ]
        for cp in cps:
            cp.wait()
        pltpu.sync_copy(rows, emb_hbm.at[pl.ds(rbase, 384)])

    @pl.when(w >= 14)
    def _():
        rbase = 42 * 128 + (w - 14) * 256
        for j in range(2):
            pltpu.sync_copy(x_hbm.at[pl.ds(rbase + j * 128, 128)],
                            idxb.at[j])
        cps = [
            pltpu.async_copy(tab_hbm.at[idxb.at[j]],
                             rows.at[pl.ds(j * 128, 128)], gsem)
            for j in range(2)
        ]
        for cp in cps:
            cp.wait()
        pltpu.sync_copy(rows.at[pl.ds(0, 256)],
                        emb_hbm.at[pl.ds(rbase, 256)])

    @pl.when(w == 31)
    def _():
        pltpu.sync_copy(x_hbm.at[pl.ds(9984, 128)], idxb.at[2])
        pltpu.async_copy(tab_hbm.at[idxb.at[2]],
                         rows.at[pl.ds(256, 128)], gsem).wait()
        pltpu.sync_copy(rows.at[pl.ds(256, 128)],
                        emb_hbm.at[pl.ds(9984, 128)])


_NPS = 156
_NGRP = 78


@functools.partial(
    pl.kernel,
    mesh=_MESH,
    out_type=[jax.ShapeDtypeStruct((2, NH, 128), jnp.float32)],
    scratch_types=[
        pltpu.VMEM((256,), jnp.int32),
        pltpu.VMEM((256,), jnp.int32),
        pltpu.VMEM((256,), jnp.int32),
        pltpu.VMEM((256,), jnp.int32),
        pltpu.VMEM((2, 128), jnp.int32),
        pltpu.VMEM((2, 128), jnp.int32),
        pltpu.VMEM((256, 128), jnp.float32),
        pltpu.VMEM((256, 128), jnp.float32),
        pltpu.VMEM((32, 128), jnp.float32),
        pltpu.VMEM_SHARED((NACC, 128), jnp.float32),
        pltpu.SemaphoreType.DMA,
        pltpu.SemaphoreType.DMA,
    ],
)
def _sc_scatter(q_hbm, a_hbm, out_hbm, srcA, srcB, dstA1, dstB1,
                dstbA, dstbB, rowsA, rowsB, zbuf, accum, gsemA, gsemB):
    c = lax.axis_index("c")
    s = lax.axis_index("s")
    cbase = c * NH
    trash = NH + s

    def fz(i, _):
        for j in range(8):
            zbuf[i, pl.ds(j * 16, 16)] = _f32z16()
        return 0

    lax.fori_loop(0, 32, fz, 0)

    @pl.when(s < 15)
    def _():
        def zc(i, _):
            pltpu.sync_copy(zbuf, accum.at[pl.ds(s * 320 + i * 32, 32)])
            return 0
        lax.fori_loop(0, 10, zc, 0)

    @pl.when(s == 15)
    def _():
        def zc(i, _):
            pltpu.sync_copy(zbuf, accum.at[pl.ds(4800 + i * 32, 32)])
            return 0
        lax.fori_loop(0, 6, zc, 0)
        pltpu.sync_copy(zbuf.at[pl.ds(0, 24)], accum.at[pl.ds(4992, 24)])

    plsc.subcore_barrier()

    def remap(dst1, dstb):
        for j in range(2):
            for k in range(8):
                d = dst1[pl.ds(j * 128 + k * 16, 16)]
                dl = d - cbase
                ok = (dl >= 0) & (dl < NH)
                dstb[j, pl.ds(k * 16, 16)] = jnp.where(ok, dl, trash)

    base = s * (_NPS * 128)

    def load_and_fire(g, srcb, dst1, rowsb, sem):
        off = base + g * 256
        pltpu.sync_copy(a_hbm.at[0].at[pl.ds(off, 256)], srcb)
        pltpu.sync_copy(a_hbm.at[1].at[pl.ds(off, 256)], dst1)
        for j in range(2):
            pltpu.async_copy(q_hbm.at[srcb.at[pl.ds(j * 128, 128)]],
                             rowsb.at[pl.ds(j * 128, 128)], sem)

    def drain(srcb, rowsb, sem):
        for j in range(2):
            pltpu.make_async_copy(q_hbm.at[srcb.at[pl.ds(j * 128, 128)]],
                                  rowsb.at[pl.ds(j * 128, 128)], sem).wait()

    def scatter(rowsb, dstb):
        for j in range(2):
            pltpu.sync_copy(rowsb.at[pl.ds(j * 128, 128)],
                            accum.at[dstb.at[j]], add=True)

    load_and_fire(0, srcA, dstA1, rowsA, gsemA)

    def body(gg, _):
        g1 = 2 * gg + 1
        load_and_fire(g1, srcB, dstB1, rowsB, gsemB)
        drain(srcA, rowsA, gsemA)
        remap(dstA1, dstbA)
        scatter(rowsA, dstbA)

        @pl.when(gg < _NGRP // 2 - 1)
        def _():
            load_and_fire(g1 + 1, srcA, dstA1, rowsA, gsemA)

        drain(srcB, rowsB, gsemB)
        remap(dstB1, dstbB)
        scatter(rowsB, dstbB)
        return 0

    lax.fori_loop(0, _NGRP // 2, body, 0)

    @pl.when(s < 4)
    def _():
        off = (2496 + s) * 128
        pltpu.sync_copy(a_hbm.at[0].at[pl.ds(off, 128)],
                        srcA.at[pl.ds(0, 128)])
        pltpu.sync_copy(a_hbm.at[1].at[pl.ds(off, 128)],
                        dstA1.at[pl.ds(0, 128)])
        cp = pltpu.async_copy(q_hbm.at[srcA.at[pl.ds(0, 128)]],
                              rowsA.at[pl.ds(0, 128)], gsemA)
        for k in range(8):
            d = dstA1[pl.ds(k * 16, 16)]
            dl = d - cbase
            ok = (dl >= 0) & (dl < NH)
            dstbA[0, pl.ds(k * 16, 16)] = jnp.where(ok, dl, trash)
        cp.wait()
        pltpu.sync_copy(rowsA.at[pl.ds(0, 128)],
                        accum.at[dstbA.at[0]], add=True)

    plsc.subcore_barrier()

    @pl.when(s < 15)
    def _():
        pltpu.sync_copy(accum.at[pl.ds(s * 312, 312)],
                        out_hbm.at[c].at[pl.ds(s * 312, 312)])

    @pl.when(s == 15)
    def _():
        pltpu.sync_copy(accum.at[pl.ds(4680, 320)],
                        out_hbm.at[c].at[pl.ds(4680, 320)])


def _leaky(v):
    return jnp.where(v > 0, v, 0.01 * v)


def _dot(x, y, precision=lax.Precision.HIGHEST):
    return lax.dot_general(x, y, (((1,), (0,)), ((), ())),
                           precision=precision,
                           preferred_element_type=jnp.float32)


_BLK = 2000


def _prep_body(degb_ref, emb_ref, dinvb_ref, q1_ref):
    dv = lax.rsqrt(degb_ref[...])
    dinvb_ref[...] = dv
    q1_ref[...] = emb_ref[...] * dv


def _prep_call(degb, emb):
    return pl.pallas_call(
        _prep_body,
        grid=(N // _BLK,),
        in_specs=[
            pl.BlockSpec((_BLK, 128), lambda i: (i, 0)),
            pl.BlockSpec((_BLK, 128), lambda i: (i, 0)),
        ],
        out_specs=[
            pl.BlockSpec((_BLK, 128), lambda i: (i, 0)),
            pl.BlockSpec((_BLK, 128), lambda i: (i, 0)),
        ],
        out_shape=[
            jax.ShapeDtypeStruct((N, 128), jnp.float32),
            jax.ShapeDtypeStruct((N, 128), jnp.float32),
        ],
    )(degb, emb)


def _mm_body(x_ref, dinvb_ref, w_ref, o_ref):
    o_ref[...] = _dot(dinvb_ref[...] * x_ref[...], w_ref[...])


def _mm_call(x, dinvb, W):
    ko = W.shape[1]
    return pl.pallas_call(
        _mm_body,
        grid=(N // _BLK,),
        in_specs=[
            pl.BlockSpec((_BLK, 128), lambda i: (i, 0)),
            pl.BlockSpec((_BLK, 128), lambda i: (i, 0)),
            pl.BlockSpec((128, ko), lambda i: (0, 0)),
        ],
        out_specs=pl.BlockSpec((_BLK, ko), lambda i: (i, 0)),
        out_shape=jax.ShapeDtypeStruct((N, ko), jnp.float32),
    )(x, dinvb, W)


def _mm2_body(s_ref, q_ref, dinvb_ref, w_ref, o_ref):
    o_ref[...] = _dot(dinvb_ref[...] * (s_ref[...] + q_ref[...]), w_ref[...])


def _mm2_call(s, q, dinvb, W):
    ko = W.shape[1]
    return pl.pallas_call(
        _mm2_body,
        grid=(N // _BLK,),
        in_specs=[
            pl.BlockSpec((_BLK, 128), lambda i: (i, 0)),
            pl.BlockSpec((_BLK, 128), lambda i: (i, 0)),
            pl.BlockSpec((_BLK, 128), lambda i: (i, 0)),
            pl.BlockSpec((128, ko), lambda i: (0, 0)),
        ],
        out_specs=pl.BlockSpec((_BLK, ko), lambda i: (i, 0)),
        out_shape=jax.ShapeDtypeStruct((N, ko), jnp.float32),
    )(s, q, dinvb, W)


def _l1fin_body(tp_ref, s_ref, dinvb_ref, w_ref, b_ref,
                t_ref, sum_ref, ssq_ref):
    t = tp_ref[...] + _dot(dinvb_ref[...] * s_ref[...], w_ref[...]) \
        + b_ref[...]
    t_ref[...] = t

    @pl.when(pl.program_id(0) == 0)
    def _():
        sum_ref[...] = jnp.zeros_like(sum_ref)
        ssq_ref[...] = jnp.zeros_like(ssq_ref)

    sum_ref[...] += jnp.sum(t, 0, keepdims=True)
    ssq_ref[...] += jnp.sum(t * t, 0, keepdims=True)


def _l1fin_call(tp, s1, dinvb, W1, b1):
    ko = W1.shape[1]
    return pl.pallas_call(
        _l1fin_body,
        grid=(N // _BLK,),
        in_specs=[
            pl.BlockSpec((_BLK, ko), lambda i: (i, 0)),
            pl.BlockSpec((_BLK, 128), lambda i: (i, 0)),
            pl.BlockSpec((_BLK, 128), lambda i: (i, 0)),
            pl.BlockSpec((128, ko), lambda i: (0, 0)),
            pl.BlockSpec((1, ko), lambda i: (0, 0)),
        ],
        out_specs=[
            pl.BlockSpec((_BLK, ko), lambda i: (i, 0)),
            pl.BlockSpec((1, ko), lambda i: (0, 0)),
            pl.BlockSpec((1, ko), lambda i: (0, 0)),
        ],
        out_shape=[
            jax.ShapeDtypeStruct((N, ko), jnp.float32),
            jax.ShapeDtypeStruct((1, ko), jnp.float32),
            jax.ShapeDtypeStruct((1, ko), jnp.float32),
        ],
    )(tp, s1, dinvb, W1, b1)


def _l2fin_body(tp_ref, s_ref, q_ref, dinvb_ref, w_ref, b_ref,
                t_ref, sum_ref, ssq_ref):
    t = tp_ref[...] + b_ref[...] + _dot(
        dinvb_ref[...] * (s_ref[...] + q_ref[...]), w_ref[...])
    t_ref[...] = t

    @pl.when(pl.program_id(0) == 0)
    def _():
        sum_ref[...] = jnp.zeros_like(sum_ref)
        ssq_ref[...] = jnp.zeros_like(ssq_ref)

    sum_ref[...] += jnp.sum(t, 0, keepdims=True)
    ssq_ref[...] += jnp.sum(t * t, 0, keepdims=True)


def _l2fin_call(tp, s2b, q2b, dinvb, W2b, b2):
    ko = W2b.shape[1]
    return pl.pallas_call(
        _l2fin_body,
        grid=(N // _BLK,),
        in_specs=[
            pl.BlockSpec((_BLK, ko), lambda i: (i, 0)),
            pl.BlockSpec((_BLK, 128), lambda i: (i, 0)),
            pl.BlockSpec((_BLK, 128), lambda i: (i, 0)),
            pl.BlockSpec((_BLK, 128), lambda i: (i, 0)),
            pl.BlockSpec((128, ko), lambda i: (0, 0)),
            pl.BlockSpec((1, ko), lambda i: (0, 0)),
        ],
        out_specs=[
            pl.BlockSpec((_BLK, ko), lambda i: (i, 0)),
            pl.BlockSpec((1, ko), lambda i: (0, 0)),
            pl.BlockSpec((1, ko), lambda i: (0, 0)),
        ],
        out_shape=[
            jax.ShapeDtypeStruct((N, ko), jnp.float32),
            jax.ShapeDtypeStruct((1, ko), jnp.float32),
            jax.ShapeDtypeStruct((1, ko), jnp.float32),
        ],
    )(tp, s2b, q2b, dinvb, W2b, b2)


def _l1p2_body(t_ref, sum_ref, ssq_ref, g_ref, be_ref, dinvb_ref,
               q2a_ref, q2b_ref):
    m = sum_ref[...] * (1.0 / N)
    v = ssq_ref[...] * (1.0 / N) - m * m
    y = (t_ref[...] - m) * (lax.rsqrt(v + 1e-5) * g_ref[...]) + be_ref[...]
    q = _leaky(y)
    dv = dinvb_ref[...]
    q2a_ref[...] = q[:, :128] * dv
    q2b_ref[...] = q[:, 128:] * dv


def _l1p2_call(t1, s1, ssq1, g1, be1, dinvb):
    return pl.pallas_call(
        _l1p2_body,
        grid=(N // _BLK,),
        in_specs=[
            pl.BlockSpec((_BLK, 256), lambda i: (i, 0)),
            pl.BlockSpec((1, 256), lambda i: (0, 0)),
            pl.BlockSpec((1, 256), lambda i: (0, 0)),
            pl.BlockSpec((1, 256), lambda i: (0, 0)),
            pl.BlockSpec((1, 256), lambda i: (0, 0)),
            pl.BlockSpec((_BLK, 128), lambda i: (i, 0)),
        ],
        out_specs=[
            pl.BlockSpec((_BLK, 128), lambda i: (i, 0)),
            pl.BlockSpec((_BLK, 128), lambda i: (i, 0)),
        ],
        out_shape=[
            jax.ShapeDtypeStruct((N, 128), jnp.float32),
            jax.ShapeDtypeStruct((N, 128), jnp.float32),
        ],
    )(t1, s1, ssq1, g1, be1, dinvb)


def _l2p2_body(t_ref, sum_ref, ssq_ref, g_ref, be_ref, h_ref):
    m = sum_ref[...] * (1.0 / N)
    v = ssq_ref[...] * (1.0 / N) - m * m
    h_ref[...] = _leaky(
        (t_ref[...] - m) * (lax.rsqrt(v + 1e-5) * g_ref[...]) + be_ref[...])


def _l2p2_call(t2, s2, ssq2, g2, be2):
    return pl.pallas_call(
        _l2p2_body,
        grid=(N // _BLK,),
        in_specs=[
            pl.BlockSpec((_BLK, 512), lambda i: (i, 0)),
            pl.BlockSpec((1, 512), lambda i: (0, 0)),
            pl.BlockSpec((1, 512), lambda i: (0, 0)),
            pl.BlockSpec((1, 512), lambda i: (0, 0)),
            pl.BlockSpec((1, 512), lambda i: (0, 0)),
        ],
        out_specs=pl.BlockSpec((_BLK, 512), lambda i: (i, 0)),
        out_shape=jax.ShapeDtypeStruct((N, 512), jnp.float32),
    )(t2, s2, ssq2, g2, be2)


_MBLK = 1000


def _mlp_body(h_ref, w1_ref, b1_ref, w2_ref, b2_ref, o_ref):
    y = _leaky(_dot(h_ref[...], w1_ref[...], None) + b1_ref[...])
    o_ref[...] = _dot(y, w2_ref[...], None) + b2_ref[...]


def _mlp_call(h, D1W, D1b, D2W, D2b):
    return pl.pallas_call(
        _mlp_body,
        grid=(N // _MBLK,),
        in_specs=[
            pl.BlockSpec((_MBLK, 512), lambda i: (i, 0)),
            pl.BlockSpec((512, 4096), lambda i: (0, 0)),
            pl.BlockSpec((1, 4096), lambda i: (0, 0)),
            pl.BlockSpec((4096, 6), lambda i: (0, 0)),
            pl.BlockSpec((1, 6), lambda i: (0, 0)),
        ],
        out_specs=pl.BlockSpec((_MBLK, 6), lambda i: (i, 0)),
        out_shape=jax.ShapeDtypeStruct((N, 6), jnp.float32),
    )(h, D1W, D1b, D2W, D2b)


def _scatter(q, a):
    s = _sc_scatter(q, a)
    if isinstance(s, (list, tuple)):
        s = s[0]
    return s.reshape(N, 128)


@jax.jit
def _pipeline(x, a, emb_table, W1, b1, g1, be1, W2, b2, g2, be2,
              D1W, D1b, D2W, D2b):
    x = x.astype(jnp.int32)
    a = a.astype(jnp.int32)
    x_pad = jnp.concatenate([x, jnp.zeros((NP - N,), jnp.int32)])
    emb, deg_parts = _sc_emb_deg(x_pad, a, emb_table)
    deg = deg_parts[0, :N] + deg_parts[1, :N] + 1.0
    degb = jnp.broadcast_to(deg[:, None], (N, 128))
    dinvb, q1 = _prep_call(degb, emb[:N])
    s1 = _scatter(q1, a)
    t1self = _mm_call(q1, dinvb, W1)
    t1, s1sum, s1ssq = _l1fin_call(t1self, s1, dinvb, W1,
                                   b1.reshape(1, -1))
    q2a, q2b = _l1p2_call(t1, s1sum, s1ssq, g1.reshape(1, -1),
                          be1.reshape(1, -1), dinvb)
    s2a = _scatter(q2a, a)
    s2b = _scatter(q2b, a)
    t2a = _mm2_call(s2a, q2a, dinvb, W2[:128])
    t2, s2sum, s2ssq = _l2fin_call(t2a, s2b, q2b, dinvb, W2[128:],
                                   b2.reshape(1, -1))
    h = _l2p2_call(t2, s2sum, s2ssq, g2.reshape(1, -1), be2.reshape(1, -1))
    return _mlp_call(h, D1W, D1b.reshape(1, -1), D2W, D2b.reshape(1, -1))


def kernel(x, a, emb_table, W1, b1, g1, be1, W2, b2, g2, be2,
           D1W, D1b, D2W, D2b):
    return _pipeline(x, a, emb_table, W1, b1, g1, be1, W2, b2, g2, be2,
                     D1W, D1b, D2W, D2b)
